# Initial kernel scaffold; baseline (speedup 1.0000x reference)
import functools
import jax
import jax.numpy as jnp
from jax import lax
from jax.experimental import pallas as pl
from jax.experimental.pallas import tpu as pltpu
from jax.experimental.pallas import tpu_sc as plsc

N = 10000
E = 160000
D = 256
H = 128
ALPHA = 0.5

NT = 16
CH = 125
EPT = E // NT
CPT = EPT // CH
RPT = N // NT
RCH = RPT // CH
UNROLL = 8

_mesh = plsc.VectorSubcoreMesh(core_axis_name="c", subcore_axis_name="s")


@functools.partial(
    pl.kernel,
    out_type=(
        jax.ShapeDtypeStruct((N, 16), jnp.float32),
        jax.ShapeDtypeStruct((N, 16), jnp.float32),
    ),
    mesh=_mesh,
    scratch_types=[
        pltpu.VMEM((CPT, CH), jnp.int32),
        pltpu.VMEM((CH, 16), jnp.float32),
        pltpu.VMEM((RPT, 16), jnp.float32),
        pltpu.VMEM_SHARED((N, 16), jnp.float32),
    ],
)
def _deg_kernel(dst2, src2, ones16, z16, deg_in, deg_out, idx_v, ones_v,
                rbuf_v, acc_sh):
    c = lax.axis_index("c")
    s = lax.axis_index("s")
    row0 = s * RPT

    pltpu.sync_copy(z16, rbuf_v)
    pltpu.sync_copy(rbuf_v, acc_sh.at[pl.ds(row0, RPT)])
    pltpu.sync_copy(ones16, ones_v)

    @pl.when(c == 0)
    def _():
        pltpu.sync_copy(dst2.at[pl.ds(s * CPT, CPT)], idx_v)

    @pl.when(c == 1)
    def _():
        pltpu.sync_copy(src2.at[pl.ds(s * CPT, CPT)], idx_v)

    plsc.subcore_barrier()

    def body(j, carry):
        for k in range(UNROLL):
            jj = j * UNROLL + k
            pltpu.sync_copy(ones_v, acc_sh.at[idx_v.at[jj]], add=True)
        return carry

    lax.fori_loop(0, CPT // UNROLL, body, 0)
    plsc.subcore_barrier()

    pltpu.sync_copy(acc_sh.at[pl.ds(row0, RPT)], rbuf_v)

    @pl.when(c == 0)
    def _():
        pltpu.sync_copy(rbuf_v, deg_in.at[pl.ds(row0, RPT)])

    @pl.when(c == 1)
    def _():
        pltpu.sync_copy(rbuf_v, deg_out.at[pl.ds(row0, RPT)])


def _dinv(deg_col):
    return jnp.where(deg_col > 0.0,
                     lax.rsqrt(jnp.maximum(deg_col, 1e-12)), 0.0)


def _matmul_scale_body(x_ref, wi_ref, wo_ref, di_ref, do_ref,
                       gil_ref, gih_ref, gol_ref, goh_ref):
    xb = x_ref[...]
    dinv_in = _dinv(di_ref[...][:, 0:1])
    dinv_out = _dinv(do_ref[...][:, 0:1])
    h_in = jnp.dot(xb, wi_ref[...], preferred_element_type=jnp.float32)
    h_out = jnp.dot(xb, wo_ref[...], preferred_element_type=jnp.float32)
    g_in = h_in * dinv_in
    g_out = h_out * dinv_out
    gil_ref[...] = g_in[:, :H]
    gih_ref[...] = g_in[:, H:]
    gol_ref[...] = g_out[:, :H]
    goh_ref[...] = g_out[:, H:]


@functools.partial(
    pl.kernel,
    out_type=(
        jax.ShapeDtypeStruct((N, H), jnp.float32),
        jax.ShapeDtypeStruct((N, H), jnp.float32),
        jax.ShapeDtypeStruct((N, H), jnp.float32),
        jax.ShapeDtypeStruct((N, H), jnp.float32),
    ),
    mesh=_mesh,
    scratch_types=[
        pltpu.VMEM((CPT, CH), jnp.int32),
        pltpu.VMEM((CPT, CH), jnp.int32),
        pltpu.VMEM((CH, H), jnp.float32),
        pltpu.VMEM((CH, H), jnp.float32),
        pltpu.VMEM_SHARED((N, H), jnp.float32),
    ],
)
def _edge_kernel(gil, gih, gol, goh, src2, dst2, z128,
                 ail, aih, aol, aoh,
                 gidx_v, sidx_v, buf_v, zbuf_v, acc_sh):
    c = lax.axis_index("c")
    s = lax.axis_index("s")
    row0 = s * RPT

    pltpu.sync_copy(z128, zbuf_v)

    def conv(g_hbm, gidx_hbm, sidx_hbm, out_hbm):
        for k in range(RCH):
            pltpu.sync_copy(zbuf_v, acc_sh.at[pl.ds(row0 + k * CH, CH)])
        pltpu.sync_copy(gidx_hbm.at[pl.ds(s * CPT, CPT)], gidx_v)
        pltpu.sync_copy(sidx_hbm.at[pl.ds(s * CPT, CPT)], sidx_v)
        plsc.subcore_barrier()

        def body(j, carry):
            for k in range(UNROLL):
                jj = j * UNROLL + k
                pltpu.sync_copy(g_hbm.at[gidx_v.at[jj]], buf_v)
                pltpu.sync_copy(buf_v, acc_sh.at[sidx_v.at[jj]], add=True)
            return carry

        lax.fori_loop(0, CPT // UNROLL, body, 0)
        plsc.subcore_barrier()

        for k in range(RCH):
            r = row0 + k * CH
            pltpu.sync_copy(acc_sh.at[pl.ds(r, CH)], buf_v)
            pltpu.sync_copy(buf_v, out_hbm.at[pl.ds(r, CH)])
        plsc.subcore_barrier()

    @pl.when(c == 0)
    def _():
        conv(gil, src2, dst2, ail)
        conv(gol, dst2, src2, aol)

    @pl.when(c == 1)
    def _():
        conv(gih, src2, dst2, aih)
        conv(goh, dst2, src2, aoh)


def _combine_body(x_ref, wr_ref, ail_ref, aih_ref, aol_ref, aoh_ref,
                  di_ref, do_ref, bi_ref, bo_ref, br_ref, out_ref):
    dinv_in = _dinv(di_ref[...][:, 0:1])
    dinv_out = _dinv(do_ref[...][:, 0:1])
    a_in = jnp.concatenate([ail_ref[...], aih_ref[...]], axis=1)
    a_out = jnp.concatenate([aol_ref[...], aoh_ref[...]], axis=1)
    root = jnp.dot(x_ref[...], wr_ref[...], preferred_element_type=jnp.float32)
    bias = (ALPHA * bo_ref[...] + (1.0 - ALPHA) * bi_ref[...] + br_ref[...])
    out_ref[...] = (ALPHA * (a_out * dinv_out)
                    + (1.0 - ALPHA) * (a_in * dinv_in)
                    + root + bias)


BR = 1000


def _row_spec(w):
    return pl.BlockSpec((BR, w), lambda i: (i, 0))


def _full_spec(shape):
    return pl.BlockSpec(shape, lambda i: tuple(0 for _ in shape))


def kernel(x, edge_index, W_in, b_in, W_out, b_out, W_root, b_root):
    src2 = edge_index[0].reshape(E // CH, CH)
    dst2 = edge_index[1].reshape(E // CH, CH)
    ones16 = jnp.ones((CH, 16), jnp.float32)
    z16 = jnp.zeros((RPT, 16), jnp.float32)
    z128 = jnp.zeros((CH, H), jnp.float32)

    deg_in, deg_out = _deg_kernel(dst2, src2, ones16, z16)

    gil, gih, gol, goh = pl.pallas_call(
        _matmul_scale_body,
        grid=(N // BR,),
        in_specs=[
            _row_spec(D), _full_spec((D, D)), _full_spec((D, D)),
            _row_spec(16), _row_spec(16),
        ],
        out_specs=[_row_spec(H)] * 4,
        out_shape=[jax.ShapeDtypeStruct((N, H), jnp.float32)] * 4,
    )(x, W_in, W_out, deg_in, deg_out)

    ail, aih, aol, aoh = _edge_kernel(gil, gih, gol, goh, src2, dst2, z128)

    out = pl.pallas_call(
        _combine_body,
        grid=(N // BR,),
        in_specs=[
            _row_spec(D), _full_spec((D, D)),
            _row_spec(H), _row_spec(H), _row_spec(H), _row_spec(H),
            _row_spec(16), _row_spec(16),
            _full_spec((1, D)), _full_spec((1, D)), _full_spec((1, D)),
        ],
        out_specs=_row_spec(D),
        out_shape=jax.ShapeDtypeStruct((N, D), jnp.float32),
    )(x, W_root, ail, aih, aol, aoh, deg_in, deg_out,
      b_in.reshape(1, D), b_out.reshape(1, D), b_root.reshape(1, D))
    return out


# speedup vs baseline: 5.8031x; 5.8031x over previous
import functools
import jax
import jax.numpy as jnp
from jax import lax
from jax.experimental import pallas as pl
from jax.experimental.pallas import tpu as pltpu
from jax.experimental.pallas import tpu_sc as plsc

N = 10000
E = 160000
D = 256
H = 128
ALPHA = 0.5

NT = 16
CH = 128
EP = 163840
NCH = EP // CH
CPT = NCH // NT
NP = 10240
RPT = NP // NT
RC = 128
RCH = RPT // RC

_mesh = plsc.VectorSubcoreMesh(core_axis_name="c", subcore_axis_name="s")


@functools.partial(
    pl.kernel,
    out_type=(
        jax.ShapeDtypeStruct((NP, H), jnp.float32),
        jax.ShapeDtypeStruct((NP, H), jnp.float32),
    ),
    mesh=_mesh,
    scratch_types=[
        pltpu.VMEM((CH,), jnp.int32),
        pltpu.VMEM((CH, H), jnp.float32),
        pltpu.VMEM((RC, H), jnp.float32),
        pltpu.VMEM_SHARED((NP, H), jnp.float32),
    ],
)
def _deg_kernel(dst2, src2, ones_hbm, z128, deg_in, deg_out, idx_v, ones_v,
                zwbuf_v, acc_sh):
    c = lax.axis_index("c")
    s = lax.axis_index("s")
    row0 = s * RPT

    pltpu.sync_copy(z128, zwbuf_v)
    for k in range(RCH):
        pltpu.sync_copy(zwbuf_v, acc_sh.at[pl.ds(row0 + k * RC, RC)])
    pltpu.sync_copy(ones_hbm, ones_v)
    plsc.subcore_barrier()

    def hist(idx_hbm):
        def body(j, carry):
            pltpu.sync_copy(idx_hbm.at[s * CPT + j], idx_v)
            pltpu.sync_copy(ones_v, acc_sh.at[idx_v], add=True)
            return carry

        lax.fori_loop(0, CPT, body, 0)

    @pl.when(c == 0)
    def _():
        hist(dst2)

    @pl.when(c == 1)
    def _():
        hist(src2)

    plsc.subcore_barrier()

    for k in range(RCH):
        r = row0 + k * RC
        pltpu.sync_copy(acc_sh.at[pl.ds(r, RC)], zwbuf_v)

        @pl.when(c == 0)
        def _():
            pltpu.sync_copy(zwbuf_v, deg_in.at[pl.ds(r, RC)])

        @pl.when(c == 1)
        def _():
            pltpu.sync_copy(zwbuf_v, deg_out.at[pl.ds(r, RC)])


def _dinv(deg_col):
    return jnp.where(deg_col > 0.0,
                     lax.rsqrt(jnp.maximum(deg_col, 1e-12)), 0.0)


def _matmul_scale_body(x_ref, wi_ref, wo_ref, di_ref, do_ref,
                       gil_ref, gih_ref, gol_ref, goh_ref):
    xb = x_ref[...]
    dinv_in = _dinv(di_ref[...][:, 0:1])
    dinv_out = _dinv(do_ref[...][:, 0:1])
    h_in = jnp.dot(xb, wi_ref[...], preferred_element_type=jnp.float32)
    h_out = jnp.dot(xb, wo_ref[...], preferred_element_type=jnp.float32)
    g_in = h_in * dinv_in
    g_out = h_out * dinv_out
    gil_ref[...] = g_in[:, :H]
    gih_ref[...] = g_in[:, H:]
    gol_ref[...] = g_out[:, :H]
    goh_ref[...] = g_out[:, H:]


@functools.partial(
    pl.kernel,
    out_type=(
        jax.ShapeDtypeStruct((NP, H), jnp.float32),
        jax.ShapeDtypeStruct((NP, H), jnp.float32),
        jax.ShapeDtypeStruct((NP, H), jnp.float32),
        jax.ShapeDtypeStruct((NP, H), jnp.float32),
    ),
    mesh=_mesh,
    scratch_types=[
        pltpu.VMEM((CH,), jnp.int32),
        pltpu.VMEM((CH,), jnp.int32),
        pltpu.VMEM((CH, H), jnp.float32),
        pltpu.VMEM((RC, H), jnp.float32),
        pltpu.VMEM_SHARED((NP, H), jnp.float32),
    ],
)
def _edge_kernel(gil, gih, gol, goh, srcg2, dstg2, srcs2, dsts2, z128,
                 ail, aih, aol, aoh,
                 gidx_v, sidx_v, ebuf_v, zwbuf_v, acc_sh):
    c = lax.axis_index("c")
    s = lax.axis_index("s")
    row0 = s * RPT

    def conv(g_hbm, gidx_hbm, sidx_hbm, out_hbm):
        pltpu.sync_copy(z128, zwbuf_v)
        for k in range(RCH):
            pltpu.sync_copy(zwbuf_v, acc_sh.at[pl.ds(row0 + k * RC, RC)])
        plsc.subcore_barrier()

        def body(j, carry):
            j0 = s * CPT + j
            pltpu.sync_copy(gidx_hbm.at[j0], gidx_v)
            pltpu.sync_copy(sidx_hbm.at[j0], sidx_v)
            pltpu.sync_copy(g_hbm.at[gidx_v], ebuf_v)
            pltpu.sync_copy(ebuf_v, acc_sh.at[sidx_v], add=True)
            return carry

        lax.fori_loop(0, CPT, body, 0)
        plsc.subcore_barrier()

        for k in range(RCH):
            r = row0 + k * RC
            pltpu.sync_copy(acc_sh.at[pl.ds(r, RC)], zwbuf_v)
            pltpu.sync_copy(zwbuf_v, out_hbm.at[pl.ds(r, RC)])
        plsc.subcore_barrier()

    @pl.when(c == 0)
    def _():
        conv(gil, srcg2, dsts2, ail)
        conv(gol, dstg2, srcs2, aol)

    @pl.when(c == 1)
    def _():
        conv(gih, srcg2, dsts2, aih)
        conv(goh, dstg2, srcs2, aoh)


def _combine_body(x_ref, wr_ref, ail_ref, aih_ref, aol_ref, aoh_ref,
                  di_ref, do_ref, bi_ref, bo_ref, br_ref, out_ref):
    dinv_in = _dinv(di_ref[...][:, 0:1])
    dinv_out = _dinv(do_ref[...][:, 0:1])
    a_in = jnp.concatenate([ail_ref[...], aih_ref[...]], axis=1)
    a_out = jnp.concatenate([aol_ref[...], aoh_ref[...]], axis=1)
    root = jnp.dot(x_ref[...], wr_ref[...], preferred_element_type=jnp.float32)
    bias = (ALPHA * bo_ref[...] + (1.0 - ALPHA) * bi_ref[...] + br_ref[...])
    out_ref[...] = (ALPHA * (a_out * dinv_out)
                    + (1.0 - ALPHA) * (a_in * dinv_in)
                    + root + bias)


BR = 1000


def _row_spec(w):
    return pl.BlockSpec((BR, w), lambda i: (i, 0))


def _full_spec(shape):
    return pl.BlockSpec(shape, lambda i: tuple(0 for _ in shape))


def kernel(x, edge_index, W_in, b_in, W_out, b_out, W_root, b_root):
    src = edge_index[0]
    dst = edge_index[1]
    pad_g = jnp.zeros((EP - E,), jnp.int32)
    pad_s = jnp.full((EP - E,), N, jnp.int32)
    srcg2 = jnp.concatenate([src, pad_g]).reshape(NCH, CH)
    srcs2 = jnp.concatenate([src, pad_s]).reshape(NCH, CH)
    dstg2 = jnp.concatenate([dst, pad_g]).reshape(NCH, CH)
    dsts2 = jnp.concatenate([dst, pad_s]).reshape(NCH, CH)
    ones128 = jnp.ones((CH, H), jnp.float32)
    z128 = jnp.zeros((RC, H), jnp.float32)

    deg_in, deg_out = _deg_kernel(dsts2, srcs2, ones128, z128)

    gil, gih, gol, goh = pl.pallas_call(
        _matmul_scale_body,
        grid=(N // BR,),
        in_specs=[
            _row_spec(D), _full_spec((D, D)), _full_spec((D, D)),
            _row_spec(H), _row_spec(H),
        ],
        out_specs=[_row_spec(H)] * 4,
        out_shape=[jax.ShapeDtypeStruct((N, H), jnp.float32)] * 4,
    )(x, W_in, W_out, deg_in, deg_out)

    ail, aih, aol, aoh = _edge_kernel(gil, gih, gol, goh,
                                      srcg2, dstg2, srcs2, dsts2, z128)

    out = pl.pallas_call(
        _combine_body,
        grid=(N // BR,),
        in_specs=[
            _row_spec(D), _full_spec((D, D)),
            _row_spec(H), _row_spec(H), _row_spec(H), _row_spec(H),
            _row_spec(H), _row_spec(H),
            _full_spec((1, D)), _full_spec((1, D)), _full_spec((1, D)),
        ],
        out_specs=_row_spec(D),
        out_shape=jax.ShapeDtypeStruct((N, D), jnp.float32),
    )(x, W_root, ail, aih, aol, aoh, deg_in, deg_out,
      b_in.reshape(1, D), b_out.reshape(1, D), b_root.reshape(1, D))
    return out


# speedup vs baseline: 7.5625x; 1.3032x over previous
import functools
import jax
import jax.numpy as jnp
from jax import lax
from jax.experimental import pallas as pl
from jax.experimental.pallas import tpu as pltpu
from jax.experimental.pallas import tpu_sc as plsc

N = 10000
E = 160000
D = 256
H = 128
ALPHA = 0.5

NT = 16
CH = 128
EP = 163840
NCH = EP // CH
CPT = NCH // NT
NP = 10240
RPT = NP // NT
RC = 128
RCH = RPT // RC

G = 16
NGRP = CPT // G
KD = 8
RGD = CPT // KD

_mesh = plsc.VectorSubcoreMesh(core_axis_name="c", subcore_axis_name="s")


@functools.partial(
    pl.kernel,
    out_type=(
        jax.ShapeDtypeStruct((NP, H), jnp.float32),
        jax.ShapeDtypeStruct((NP, H), jnp.float32),
    ),
    mesh=_mesh,
    scratch_types=[
        pltpu.VMEM((CPT, CH), jnp.int32),
        pltpu.VMEM((CH, H), jnp.float32),
        pltpu.VMEM_SHARED((NP, H), jnp.float32),
        pltpu.SemaphoreType.DMA,
    ],
)
def _deg_kernel(dst2, src2, ones_hbm, z128, deg_in, deg_out, idx_all_v,
                buf_v, acc_sh, sem_d):
    c = lax.axis_index("c")
    s = lax.axis_index("s")
    row0 = s * RPT

    pltpu.sync_copy(z128, buf_v)
    for k in range(RCH):
        pltpu.sync_copy(buf_v, acc_sh.at[pl.ds(row0 + k * RC, RC)])
    pltpu.sync_copy(ones_hbm, buf_v)
    plsc.subcore_barrier()

    def hist(idx_hbm):
        pltpu.sync_copy(idx_hbm.at[pl.ds(s * CPT, CPT)], idx_all_v)

        def fire(j):
            pltpu.async_copy(buf_v, acc_sh.at[idx_all_v.at[j]], sem_d,
                             add=True)

        def drain1():
            pltpu.make_async_copy(ones_hbm, buf_v, sem_d).wait()

        for b in range(KD):
            fire(b)

        def body(r, carry):
            for b in range(KD):
                drain1()
                fire(r * KD + b)
            return carry

        lax.fori_loop(1, RGD, body, 0)
        for b in range(KD):
            drain1()

    @pl.when(c == 0)
    def _():
        hist(dst2)

    @pl.when(c == 1)
    def _():
        hist(src2)

    plsc.subcore_barrier()

    for k in range(RCH):
        r = row0 + k * RC
        pltpu.sync_copy(acc_sh.at[pl.ds(r, RC)], buf_v)

        @pl.when(c == 0)
        def _():
            pltpu.sync_copy(buf_v, deg_in.at[pl.ds(r, RC)])

        @pl.when(c == 1)
        def _():
            pltpu.sync_copy(buf_v, deg_out.at[pl.ds(r, RC)])


def _dinv(deg_col):
    return jnp.where(deg_col > 0.0,
                     lax.rsqrt(jnp.maximum(deg_col, 1e-12)), 0.0)


def _matmul_scale_body(x_ref, wi_ref, wo_ref, di_ref, do_ref,
                       gil_ref, gih_ref, gol_ref, goh_ref):
    xb = x_ref[...]
    dinv_in = _dinv(di_ref[...][:, 0:1])
    dinv_out = _dinv(do_ref[...][:, 0:1])
    h_in = jnp.dot(xb, wi_ref[...], preferred_element_type=jnp.float32)
    h_out = jnp.dot(xb, wo_ref[...], preferred_element_type=jnp.float32)
    g_in = h_in * dinv_in
    g_out = h_out * dinv_out
    gil_ref[...] = g_in[:, :H]
    gih_ref[...] = g_in[:, H:]
    gol_ref[...] = g_out[:, :H]
    goh_ref[...] = g_out[:, H:]


@functools.partial(
    pl.kernel,
    out_type=(
        jax.ShapeDtypeStruct((NP, H), jnp.float32),
        jax.ShapeDtypeStruct((NP, H), jnp.float32),
        jax.ShapeDtypeStruct((NP, H), jnp.float32),
        jax.ShapeDtypeStruct((NP, H), jnp.float32),
    ),
    mesh=_mesh,
    scratch_types=[
        pltpu.VMEM((2, G, CH), jnp.int32),
        pltpu.VMEM((2, G, CH), jnp.int32),
        pltpu.VMEM((2, CH, H), jnp.float32),
        pltpu.VMEM_SHARED((NP, H), jnp.float32),
    ] + [pltpu.SemaphoreType.DMA] * 6,
)
def _edge_kernel(gil, gih, gol, goh, srcg2, dstg2, srcs2, dsts2, z128,
                 ail, aih, aol, aoh,
                 gidxb_v, sidxb_v, ebuf_v, acc_sh,
                 sem_i0, sem_i1, sem_g0, sem_g1, sem_s0, sem_s1):
    c = lax.axis_index("c")
    s = lax.axis_index("s")
    row0 = s * RPT
    sem_i = (sem_i0, sem_i1)
    sem_g = (sem_g0, sem_g1)
    sem_s = (sem_s0, sem_s1)

    def conv(g_hbm, gidx_hbm, sidx_hbm, out_hbm):
        base = s * CPT

        def fire_idx(grp, isl):
            pltpu.async_copy(gidx_hbm.at[pl.ds(base + grp * G, G)],
                             gidxb_v.at[isl], sem_i[isl])
            pltpu.async_copy(sidx_hbm.at[pl.ds(base + grp * G, G)],
                             sidxb_v.at[isl], sem_i[isl])

        def wait_idx(isl):
            pltpu.make_async_copy(gidx_hbm.at[pl.ds(0, G)],
                                  gidxb_v.at[isl], sem_i[isl]).wait()
            pltpu.make_async_copy(gidx_hbm.at[pl.ds(0, G)],
                                  sidxb_v.at[isl], sem_i[isl]).wait()

        def fire_g(isl, q, b):
            pltpu.async_copy(g_hbm.at[gidxb_v.at[isl, q]], ebuf_v.at[b],
                             sem_g[b])

        def wait_g(b):
            pltpu.make_async_copy(z128, ebuf_v.at[b], sem_g[b]).wait()

        def fire_s(isl, q, b):
            pltpu.async_copy(ebuf_v.at[b], acc_sh.at[sidxb_v.at[isl, q]],
                             sem_s[b], add=True)

        def wait_s(b):
            pltpu.make_async_copy(z128, ebuf_v.at[b], sem_s[b]).wait()

        pltpu.sync_copy(z128, ebuf_v.at[0])
        hz = [pltpu.async_copy(ebuf_v.at[0],
                               acc_sh.at[pl.ds(row0 + k * RC, RC)],
                               sem_g[0])
              for k in range(RCH)]
        for hh in hz:
            hh.wait()
        plsc.subcore_barrier()

        fire_idx(0, 0)
        fire_idx(1, 1)
        wait_idx(0)
        fire_g(0, 0, 0)

        for grp in range(NGRP):
            isl = grp & 1
            wait_g(0)
            fire_s(isl, 0, 0)
            if grp > 0:
                wait_s(1)
            fire_g(isl, 1, 1)
            wait_g(1)
            fire_s(isl, 1, 1)
            wait_s(0)
            fire_g(isl, 2, 0)

            def pair_body(p, carry):
                q0 = 2 + 2 * p
                wait_g(0)
                fire_s(isl, q0, 0)
                wait_s(1)
                fire_g(isl, q0 + 1, 1)
                wait_g(1)
                fire_s(isl, q0 + 1, 1)
                wait_s(0)
                fire_g(isl, q0 + 2, 0)
                return carry

            lax.fori_loop(0, (G - 4) // 2, pair_body, 0)

            wait_g(0)
            fire_s(isl, G - 2, 0)
            wait_s(1)
            fire_g(isl, G - 1, 1)
            wait_g(1)
            fire_s(isl, G - 1, 1)
            wait_s(0)
            if grp + 1 < NGRP:
                wait_idx(1 - isl)
                fire_g(1 - isl, 0, 0)
                if grp + 2 < NGRP:
                    fire_idx(grp + 2, isl)
        wait_s(1)
        plsc.subcore_barrier()

        def rb_read(k, b):
            return pltpu.async_copy(acc_sh.at[pl.ds(row0 + k * RC, RC)],
                                    ebuf_v.at[b], sem_g[b])

        def rb_write(k, b):
            pltpu.async_copy(ebuf_v.at[b],
                             out_hbm.at[pl.ds(row0 + k * RC, RC)], sem_s[b])

        rb_read(0, 0)
        rb_read(1, 1)
        wait_g(0)
        rb_write(0, 0)
        wait_g(1)
        rb_write(1, 1)
        for k in range(2, RCH):
            b = k & 1
            wait_s(b)
            rb_read(k, b)
            wait_g(b)
            rb_write(k, b)
        wait_s(RCH & 1)
        wait_s(1 - (RCH & 1))
        plsc.subcore_barrier()

    @pl.when(c == 0)
    def _():
        conv(gil, srcg2, dsts2, ail)
        conv(gol, dstg2, srcs2, aol)

    @pl.when(c == 1)
    def _():
        conv(gih, srcg2, dsts2, aih)
        conv(goh, dstg2, srcs2, aoh)


def _combine_body(x_ref, wr_ref, ail_ref, aih_ref, aol_ref, aoh_ref,
                  di_ref, do_ref, bi_ref, bo_ref, br_ref, out_ref):
    dinv_in = _dinv(di_ref[...][:, 0:1])
    dinv_out = _dinv(do_ref[...][:, 0:1])
    a_in = jnp.concatenate([ail_ref[...], aih_ref[...]], axis=1)
    a_out = jnp.concatenate([aol_ref[...], aoh_ref[...]], axis=1)
    root = jnp.dot(x_ref[...], wr_ref[...], preferred_element_type=jnp.float32)
    bias = (ALPHA * bo_ref[...] + (1.0 - ALPHA) * bi_ref[...] + br_ref[...])
    out_ref[...] = (ALPHA * (a_out * dinv_out)
                    + (1.0 - ALPHA) * (a_in * dinv_in)
                    + root + bias)


BR = 1000


def _row_spec(w):
    return pl.BlockSpec((BR, w), lambda i: (i, 0))


def _full_spec(shape):
    return pl.BlockSpec(shape, lambda i: tuple(0 for _ in shape))


def kernel(x, edge_index, W_in, b_in, W_out, b_out, W_root, b_root):
    src = edge_index[0]
    dst = edge_index[1]
    pad_g = jnp.zeros((EP - E,), jnp.int32)
    pad_s = jnp.full((EP - E,), N, jnp.int32)
    srcg2 = jnp.concatenate([src, pad_g]).reshape(NCH, CH)
    srcs2 = jnp.concatenate([src, pad_s]).reshape(NCH, CH)
    dstg2 = jnp.concatenate([dst, pad_g]).reshape(NCH, CH)
    dsts2 = jnp.concatenate([dst, pad_s]).reshape(NCH, CH)
    ones128 = jnp.ones((CH, H), jnp.float32)
    z128 = jnp.zeros((RC, H), jnp.float32)

    deg_in, deg_out = _deg_kernel(dsts2, srcs2, ones128, z128)

    gil, gih, gol, goh = pl.pallas_call(
        _matmul_scale_body,
        grid=(N // BR,),
        in_specs=[
            _row_spec(D), _full_spec((D, D)), _full_spec((D, D)),
            _row_spec(H), _row_spec(H),
        ],
        out_specs=[_row_spec(H)] * 4,
        out_shape=[jax.ShapeDtypeStruct((N, H), jnp.float32)] * 4,
    )(x, W_in, W_out, deg_in, deg_out)

    ail, aih, aol, aoh = _edge_kernel(gil, gih, gol, goh,
                                      srcg2, dstg2, srcs2, dsts2, z128)

    out = pl.pallas_call(
        _combine_body,
        grid=(N // BR,),
        in_specs=[
            _row_spec(D), _full_spec((D, D)),
            _row_spec(H), _row_spec(H), _row_spec(H), _row_spec(H),
            _row_spec(H), _row_spec(H),
            _full_spec((1, D)), _full_spec((1, D)), _full_spec((1, D)),
        ],
        out_specs=_row_spec(D),
        out_shape=jax.ShapeDtypeStruct((N, D), jnp.float32),
    )(x, W_root, ail, aih, aol, aoh, deg_in, deg_out,
      b_in.reshape(1, D), b_out.reshape(1, D), b_root.reshape(1, D))
    return out
